# Initial kernel scaffold; baseline (speedup 1.0000x reference)
import functools

import jax
import jax.numpy as jnp
from jax import lax
from jax.experimental import pallas as pl
from jax.experimental.pallas import tpu as pltpu
from jax.experimental.pallas import tpu_sc as plsc

N = 10000
D = 128
H = 16
OUT = 128
E = 320000

NC = 2
NS = 16
NW = NC * NS
CHUNK = 128
CB = 79
EPAD = NW * CB * CHUNK
NPAD = 11264
TRASH = NPAD - N
RS_ACC = NPAD // NS
RS_G = N // NS

_mesh = plsc.VectorSubcoreMesh(core_axis_name="c", subcore_axis_name="s")


def _deg_body(dstR, zeros1, out, dst_v, ones_v, deg_s):
    cid = lax.axis_index("c")
    sid = lax.axis_index("s")
    w = cid * NS + sid
    pltpu.sync_copy(zeros1.at[pl.ds(sid * RS_ACC, RS_ACC)],
                    deg_s.at[pl.ds(sid * RS_ACC, RS_ACC)])
    pltpu.sync_copy(dstR.at[w], dst_v)
    for j in range(CHUNK // 16):
        ones_v[pl.ds(j * 16, 16)] = jnp.ones((16,), jnp.float32)
    plsc.subcore_barrier()

    def step(c, carry):
        pltpu.sync_copy(ones_v, deg_s.at[dst_v.at[c]], add=True)
        return carry

    lax.fori_loop(0, CB, step, 0)
    plsc.subcore_barrier()
    pltpu.sync_copy(deg_s.at[pl.ds(sid * RS_ACC, RS_ACC)],
                    out.at[cid, pl.ds(sid * RS_ACC, RS_ACC)])


_deg = pl.kernel(
    _deg_body,
    out_type=jax.ShapeDtypeStruct((NC, NPAD), jnp.float32),
    mesh=_mesh,
    scratch_types=[
        pltpu.VMEM((CB, CHUNK), jnp.int32),
        pltpu.VMEM((CHUNK,), jnp.float32),
        pltpu.VMEM_SHARED((NPAD,), jnp.float32),
    ],
)


def _hop_body(g_hbm, srcR, dstR, zeros2, out, src_v, dst_v, rows_v, g_s, acc_s):
    cid = lax.axis_index("c")
    sid = lax.axis_index("s")
    w = cid * NS + sid
    pltpu.sync_copy(g_hbm.at[pl.ds(sid * RS_G, RS_G), :],
                    g_s.at[pl.ds(sid * RS_G, RS_G), :])
    pltpu.sync_copy(zeros2.at[pl.ds(sid * RS_ACC, RS_ACC), :],
                    acc_s.at[pl.ds(sid * RS_ACC, RS_ACC), :])
    pltpu.sync_copy(srcR.at[w], src_v)
    pltpu.sync_copy(dstR.at[w], dst_v)
    plsc.subcore_barrier()

    def step(c, carry):
        pltpu.sync_copy(g_s.at[src_v.at[c]], rows_v)
        pltpu.sync_copy(rows_v, acc_s.at[dst_v.at[c]], add=True)
        return carry

    lax.fori_loop(0, CB, step, 0)
    plsc.subcore_barrier()
    pltpu.sync_copy(acc_s.at[pl.ds(sid * RS_ACC, RS_ACC), :],
                    out.at[cid, pl.ds(sid * RS_ACC, RS_ACC), :])


_hop = pl.kernel(
    _hop_body,
    out_type=jax.ShapeDtypeStruct((NC, NPAD, H), jnp.float32),
    mesh=_mesh,
    scratch_types=[
        pltpu.VMEM((CB, CHUNK), jnp.int32),
        pltpu.VMEM((CB, CHUNK), jnp.int32),
        pltpu.VMEM((CHUNK, H), jnp.float32),
        pltpu.VMEM_SHARED((N, H), jnp.float32),
        pltpu.VMEM_SHARED((NPAD, H), jnp.float32),
    ],
)


_RB = 1000
_GRID = N // _RB


def _prep_body(x_ref, w_ref, p0_ref, p1_ref, g1_ref, dis_ref, dis2_ref):
    deg = p0_ref[...] + p1_ref[...] + 1.0
    dis = lax.rsqrt(deg)
    h0 = jnp.dot(x_ref[...], w_ref[...], preferred_element_type=jnp.float32)
    g1_ref[...] = dis * h0
    dis_ref[...] = dis
    dis2_ref[...] = dis * dis


def _tc_prep(x, W_conv, p0, p1):
    return pl.pallas_call(
        _prep_body,
        grid=(_GRID,),
        in_specs=[
            pl.BlockSpec((_RB, D), lambda i: (i, 0)),
            pl.BlockSpec((D, H), lambda i: (0, 0)),
            pl.BlockSpec((_RB, 1), lambda i: (i, 0)),
            pl.BlockSpec((_RB, 1), lambda i: (i, 0)),
        ],
        out_specs=[
            pl.BlockSpec((_RB, H), lambda i: (i, 0)),
            pl.BlockSpec((_RB, 1), lambda i: (i, 0)),
            pl.BlockSpec((_RB, 1), lambda i: (i, 0)),
        ],
        out_shape=[
            jax.ShapeDtypeStruct((N, H), jnp.float32),
            jax.ShapeDtypeStruct((N, 1), jnp.float32),
            jax.ShapeDtypeStruct((N, 1), jnp.float32),
        ],
    )(x, W_conv, p0, p1)


def _mid_body(p0_ref, p1_ref, g1_ref, dis2_ref, g2_ref):
    g2_ref[...] = dis2_ref[...] * (p0_ref[...] + p1_ref[...] + g1_ref[...])


def _tc_mid(p0, p1, g1, dis2):
    return pl.pallas_call(
        _mid_body,
        grid=(_GRID,),
        in_specs=[
            pl.BlockSpec((_RB, H), lambda i: (i, 0)),
            pl.BlockSpec((_RB, H), lambda i: (i, 0)),
            pl.BlockSpec((_RB, H), lambda i: (i, 0)),
            pl.BlockSpec((_RB, 1), lambda i: (i, 0)),
        ],
        out_specs=pl.BlockSpec((_RB, H), lambda i: (i, 0)),
        out_shape=jax.ShapeDtypeStruct((N, H), jnp.float32),
    )(p0, p1, g1, dis2)


def _out_body(q0_ref, q1_ref, g2_ref, dis_ref, bc_ref, wl_ref, bl_ref, out_ref):
    h2 = dis_ref[...] * (q0_ref[...] + q1_ref[...] + g2_ref[...])
    a = jnp.maximum(h2 + bc_ref[...], 0.0)
    out_ref[...] = (jnp.dot(a, wl_ref[...], preferred_element_type=jnp.float32)
                    + bl_ref[...])


def _tc_out(q0, q1, g2, dis, bc, wl, bl):
    return pl.pallas_call(
        _out_body,
        grid=(_GRID,),
        in_specs=[
            pl.BlockSpec((_RB, H), lambda i: (i, 0)),
            pl.BlockSpec((_RB, H), lambda i: (i, 0)),
            pl.BlockSpec((_RB, H), lambda i: (i, 0)),
            pl.BlockSpec((_RB, 1), lambda i: (i, 0)),
            pl.BlockSpec((1, H), lambda i: (0, 0)),
            pl.BlockSpec((H, OUT), lambda i: (0, 0)),
            pl.BlockSpec((1, OUT), lambda i: (0, 0)),
        ],
        out_specs=pl.BlockSpec((_RB, OUT), lambda i: (i, 0)),
        out_shape=jax.ShapeDtypeStruct((N, OUT), jnp.float32),
    )(q0, q1, g2, dis, bc, wl, bl)


def kernel(x, edge_index, W_conv, b_conv, W_lin, b_lin):
    src = edge_index[0]
    dst = edge_index[1]
    npad_e = EPAD - E
    pad_i = jnp.arange(npad_e, dtype=jnp.int32)
    pad_src = (pad_i * 97) % N
    pad_dst = N + (pad_i % TRASH)
    srcR = jnp.concatenate([src, pad_src]).reshape(NW, CB, CHUNK)
    dstR = jnp.concatenate([dst, pad_dst]).reshape(NW, CB, CHUNK)
    zeros1 = jnp.zeros((NPAD,), jnp.float32)
    zeros2 = jnp.zeros((NPAD, H), jnp.float32)

    degP = _deg(dstR, zeros1)
    p0 = degP[0, :N].reshape(N, 1)
    p1 = degP[1, :N].reshape(N, 1)
    g1, dis, dis2 = _tc_prep(x, W_conv, p0, p1)

    P = _hop(g1, srcR, dstR, zeros2)
    g2 = _tc_mid(P[0, :N], P[1, :N], g1, dis2)

    Q = _hop(g2, srcR, dstR, zeros2)
    out = _tc_out(Q[0, :N], Q[1, :N], g2, dis,
                  b_conv.reshape(1, H), W_lin, b_lin.reshape(1, OUT))
    return out


# speedup vs baseline: 34.3156x; 34.3156x over previous
import functools

import jax
import jax.numpy as jnp
from jax import lax
from jax.experimental import pallas as pl
from jax.experimental.pallas import tpu as pltpu
from jax.experimental.pallas import tpu_sc as plsc

N = 10000
D = 128
H = 16
OUT = 128
E = 320000

NC = 2
NS = 16
NW = NC * NS
CHUNK = 128
CB = 79
EPAD = NW * CB * CHUNK
NPAD = 10112
TRASH = NPAD - N
RS_ACC = NPAD // NS

_mesh = plsc.VectorSubcoreMesh(core_axis_name="c", subcore_axis_name="s")


def _deg_body(dstR, out, dst_v, ones_v, tmp_v, deg_s):
    cid = lax.axis_index("c")
    sid = lax.axis_index("s")
    w = cid * NS + sid
    for j in range(RS_ACC // 16):
        tmp_v[pl.ds(j * 16, 16)] = jnp.zeros((16,), jnp.float32)
    pltpu.sync_copy(tmp_v, deg_s.at[pl.ds(sid * RS_ACC, RS_ACC)])
    pltpu.sync_copy(dstR.at[w], dst_v)
    for j in range(CHUNK // 16):
        ones_v[pl.ds(j * 16, 16)] = jnp.ones((16,), jnp.float32)
    plsc.subcore_barrier()

    def step(c, carry):
        pltpu.sync_copy(ones_v, deg_s.at[dst_v.at[c]], add=True)
        return carry

    lax.fori_loop(0, CB, step, 0)
    plsc.subcore_barrier()
    pltpu.sync_copy(deg_s.at[pl.ds(sid * RS_ACC, RS_ACC)], tmp_v)
    pltpu.sync_copy(tmp_v, out.at[pl.ds(cid * NPAD + sid * RS_ACC, RS_ACC)])


_deg = pl.kernel(
    _deg_body,
    out_type=jax.ShapeDtypeStruct((NC * NPAD,), jnp.float32),
    mesh=_mesh,
    scratch_types=[
        pltpu.VMEM((CB, CHUNK), jnp.int32),
        pltpu.VMEM((CHUNK,), jnp.float32),
        pltpu.VMEM((RS_ACC,), jnp.float32),
        pltpu.VMEM_SHARED((NPAD,), jnp.float32),
    ],
)


def _hop_body(g_hbm, srcR, dstR, out, src_v, dst_v, rows_v, tmp_v, acc_s):
    cid = lax.axis_index("c")
    sid = lax.axis_index("s")
    w = cid * NS + sid
    def zstep(j, carry):
        tmp_v[j] = jnp.zeros((16,), jnp.float32)
        return carry

    lax.fori_loop(0, RS_ACC, zstep, 0)
    pltpu.sync_copy(tmp_v, acc_s.at[pl.ds(sid * RS_ACC, RS_ACC), :])
    pltpu.sync_copy(srcR.at[w], src_v)
    pltpu.sync_copy(dstR.at[w], dst_v)
    plsc.subcore_barrier()

    def step(c, carry):
        pltpu.sync_copy(g_hbm.at[src_v.at[c]], rows_v)
        pltpu.sync_copy(rows_v, acc_s.at[dst_v.at[c]], add=True)
        return carry

    lax.fori_loop(0, CB, step, 0)
    plsc.subcore_barrier()
    pltpu.sync_copy(acc_s.at[pl.ds(sid * RS_ACC, RS_ACC), :], tmp_v)
    pltpu.sync_copy(tmp_v, out.at[pl.ds(cid * NPAD + sid * RS_ACC, RS_ACC), :])


_hop = pl.kernel(
    _hop_body,
    out_type=jax.ShapeDtypeStruct((NC * NPAD, H), jnp.float32),
    mesh=_mesh,
    scratch_types=[
        pltpu.VMEM((CB, CHUNK), jnp.int32),
        pltpu.VMEM((CB, CHUNK), jnp.int32),
        pltpu.VMEM((CHUNK, H), jnp.float32),
        pltpu.VMEM((RS_ACC, H), jnp.float32),
        pltpu.VMEM_SHARED((NPAD, H), jnp.float32),
    ],
    compiler_params=pltpu.CompilerParams(use_tc_tiling_on_sc=False),
)


_RB = 1000
_GRID = N // _RB


def _prep_body(x_ref, w_ref, p0_ref, p1_ref, g1_ref, dis_ref, dis2_ref):
    deg = p0_ref[...] + p1_ref[...] + 1.0
    dis = lax.rsqrt(deg)
    h0 = jnp.dot(x_ref[...], w_ref[...], preferred_element_type=jnp.float32)
    g1_ref[...] = dis * h0
    dis_ref[...] = dis
    dis2_ref[...] = dis * dis


def _tc_prep(x, W_conv, p0, p1):
    return pl.pallas_call(
        _prep_body,
        grid=(_GRID,),
        in_specs=[
            pl.BlockSpec((_RB, D), lambda i: (i, 0)),
            pl.BlockSpec((D, H), lambda i: (0, 0)),
            pl.BlockSpec((_RB, 1), lambda i: (i, 0)),
            pl.BlockSpec((_RB, 1), lambda i: (i, 0)),
        ],
        out_specs=[
            pl.BlockSpec((_RB, H), lambda i: (i, 0)),
            pl.BlockSpec((_RB, 1), lambda i: (i, 0)),
            pl.BlockSpec((_RB, 1), lambda i: (i, 0)),
        ],
        out_shape=[
            jax.ShapeDtypeStruct((N, H), jnp.float32),
            jax.ShapeDtypeStruct((N, 1), jnp.float32),
            jax.ShapeDtypeStruct((N, 1), jnp.float32),
        ],
    )(x, W_conv, p0, p1)


def _mid_body(p0_ref, p1_ref, g1_ref, dis2_ref, g2_ref):
    g2_ref[...] = dis2_ref[...] * (p0_ref[...] + p1_ref[...] + g1_ref[...])


def _tc_mid(p0, p1, g1, dis2):
    return pl.pallas_call(
        _mid_body,
        grid=(_GRID,),
        in_specs=[
            pl.BlockSpec((_RB, H), lambda i: (i, 0)),
            pl.BlockSpec((_RB, H), lambda i: (i, 0)),
            pl.BlockSpec((_RB, H), lambda i: (i, 0)),
            pl.BlockSpec((_RB, 1), lambda i: (i, 0)),
        ],
        out_specs=pl.BlockSpec((_RB, H), lambda i: (i, 0)),
        out_shape=jax.ShapeDtypeStruct((N, H), jnp.float32),
    )(p0, p1, g1, dis2)


def _out_body(q0_ref, q1_ref, g2_ref, dis_ref, bc_ref, wl_ref, bl_ref, out_ref):
    h2 = dis_ref[...] * (q0_ref[...] + q1_ref[...] + g2_ref[...])
    a = jnp.maximum(h2 + bc_ref[...], 0.0)
    out_ref[...] = (jnp.dot(a, wl_ref[...], preferred_element_type=jnp.float32)
                    + bl_ref[...])


def _tc_out(q0, q1, g2, dis, bc, wl, bl):
    return pl.pallas_call(
        _out_body,
        grid=(_GRID,),
        in_specs=[
            pl.BlockSpec((_RB, H), lambda i: (i, 0)),
            pl.BlockSpec((_RB, H), lambda i: (i, 0)),
            pl.BlockSpec((_RB, H), lambda i: (i, 0)),
            pl.BlockSpec((_RB, 1), lambda i: (i, 0)),
            pl.BlockSpec((1, H), lambda i: (0, 0)),
            pl.BlockSpec((H, OUT), lambda i: (0, 0)),
            pl.BlockSpec((1, OUT), lambda i: (0, 0)),
        ],
        out_specs=pl.BlockSpec((_RB, OUT), lambda i: (i, 0)),
        out_shape=jax.ShapeDtypeStruct((N, OUT), jnp.float32),
    )(q0, q1, g2, dis, bc, wl, bl)


def kernel(x, edge_index, W_conv, b_conv, W_lin, b_lin):
    src = edge_index[0]
    dst = edge_index[1]
    npad_e = EPAD - E
    pad_i = jnp.arange(npad_e, dtype=jnp.int32)
    pad_src = (pad_i * 97) % N
    pad_dst = N + (pad_i % TRASH)
    srcR = jnp.concatenate([src, pad_src]).reshape(NW, CB, CHUNK)
    dstR = jnp.concatenate([dst, pad_dst]).reshape(NW, CB, CHUNK)

    degP = _deg(dstR)
    p0 = degP[:N].reshape(N, 1)
    p1 = degP[NPAD:NPAD + N].reshape(N, 1)
    g1, dis, dis2 = _tc_prep(x, W_conv, p0, p1)

    P = _hop(g1, srcR, dstR)
    g2 = _tc_mid(P[:N], P[NPAD:NPAD + N], g1, dis2)

    Q = _hop(g2, srcR, dstR)
    out = _tc_out(Q[:N], Q[NPAD:NPAD + N], g2, dis,
                  b_conv.reshape(1, H), W_lin, b_lin.reshape(1, OUT))
    return out


# speedup vs baseline: 44.7047x; 1.3028x over previous
import functools

import jax
import jax.numpy as jnp
from jax import lax
from jax.experimental import pallas as pl
from jax.experimental.pallas import tpu as pltpu
from jax.experimental.pallas import tpu_sc as plsc

N = 10000
D = 128
H = 16
OUT = 128
E = 320000

NC = 2
NS = 16
NW = NC * NS
CHUNK = 128
CB = 80
EPAD = NW * CB * CHUNK
NPAD = 10112
TRASH = NPAD - N
RS_ACC = NPAD // NS

_mesh = plsc.VectorSubcoreMesh(core_axis_name="c", subcore_axis_name="s")


def _deg_body(dstR, out, dst_v, ones_v, tmp_v, deg_s):
    cid = lax.axis_index("c")
    sid = lax.axis_index("s")
    w = cid * NS + sid
    for j in range(RS_ACC // 16):
        tmp_v[pl.ds(j * 16, 16)] = jnp.zeros((16,), jnp.float32)
    pltpu.sync_copy(tmp_v, deg_s.at[pl.ds(sid * RS_ACC, RS_ACC)])
    pltpu.sync_copy(dstR.at[w], dst_v)
    for j in range(CHUNK // 16):
        ones_v[pl.ds(j * 16, 16)] = jnp.ones((16,), jnp.float32)
    plsc.subcore_barrier()

    def step(c, carry):
        pltpu.sync_copy(ones_v, deg_s.at[dst_v.at[c]], add=True)
        return carry

    lax.fori_loop(0, CB, step, 0)
    plsc.subcore_barrier()
    pltpu.sync_copy(deg_s.at[pl.ds(sid * RS_ACC, RS_ACC)], tmp_v)
    pltpu.sync_copy(tmp_v, out.at[pl.ds(cid * NPAD + sid * RS_ACC, RS_ACC)])


_deg = pl.kernel(
    _deg_body,
    out_type=jax.ShapeDtypeStruct((NC * NPAD,), jnp.float32),
    mesh=_mesh,
    scratch_types=[
        pltpu.VMEM((CB, CHUNK), jnp.int32),
        pltpu.VMEM((CHUNK,), jnp.float32),
        pltpu.VMEM((RS_ACC,), jnp.float32),
        pltpu.VMEM_SHARED((NPAD,), jnp.float32),
    ],
)


def _hop_body(g_hbm, srcR, dstR, out, src_v, dst_v, bufa_v, bufb_v, tmp_v, acc_s,
              sema, semb):
    cid = lax.axis_index("c")
    sid = lax.axis_index("s")
    w = cid * NS + sid
    def zstep(j, carry):
        tmp_v[j] = jnp.zeros((16,), jnp.float32)
        return carry

    lax.fori_loop(0, RS_ACC, zstep, 0)
    pltpu.sync_copy(tmp_v, acc_s.at[pl.ds(sid * RS_ACC, RS_ACC), :])
    pltpu.sync_copy(srcR.at[w], src_v)
    pltpu.sync_copy(dstR.at[w], dst_v)
    plsc.subcore_barrier()

    def gath(c, buf, sem):
        return pltpu.async_copy(g_hbm.at[src_v.at[c]], buf, sem)

    def scat(c, buf):
        pltpu.sync_copy(buf, acc_s.at[dst_v.at[c]], add=True)

    gath(0, bufa_v, sema)

    def step(i, carry):
        c = 2 * i
        gath(c + 1, bufb_v, semb)
        pltpu.make_async_copy(g_hbm.at[src_v.at[c]], bufa_v, sema).wait()
        scat(c, bufa_v)
        gath(c + 2, bufa_v, sema)
        pltpu.make_async_copy(g_hbm.at[src_v.at[c]], bufb_v, semb).wait()
        scat(c + 1, bufb_v)
        return carry

    lax.fori_loop(0, CB // 2 - 1, step, 0)
    gath(CB - 1, bufb_v, semb)
    pltpu.make_async_copy(g_hbm.at[src_v.at[0]], bufa_v, sema).wait()
    scat(CB - 2, bufa_v)
    pltpu.make_async_copy(g_hbm.at[src_v.at[0]], bufb_v, semb).wait()
    scat(CB - 1, bufb_v)

    plsc.subcore_barrier()
    pltpu.sync_copy(acc_s.at[pl.ds(sid * RS_ACC, RS_ACC), :], tmp_v)
    pltpu.sync_copy(tmp_v, out.at[pl.ds(cid * NPAD + sid * RS_ACC, RS_ACC), :])


_hop = pl.kernel(
    _hop_body,
    out_type=jax.ShapeDtypeStruct((NC * NPAD, H), jnp.float32),
    mesh=_mesh,
    scratch_types=[
        pltpu.VMEM((CB, CHUNK), jnp.int32),
        pltpu.VMEM((CB, CHUNK), jnp.int32),
        pltpu.VMEM((CHUNK, H), jnp.float32),
        pltpu.VMEM((CHUNK, H), jnp.float32),
        pltpu.VMEM((RS_ACC, H), jnp.float32),
        pltpu.VMEM_SHARED((NPAD, H), jnp.float32),
        pltpu.SemaphoreType.DMA,
        pltpu.SemaphoreType.DMA,
    ],
    compiler_params=pltpu.CompilerParams(use_tc_tiling_on_sc=False),
)


_RB = 1000
_GRID = N // _RB


def _prep_body(x_ref, w_ref, p0_ref, p1_ref, g1_ref, dis_ref, dis2_ref):
    deg = p0_ref[...] + p1_ref[...] + 1.0
    dis = lax.rsqrt(deg)
    h0 = jnp.dot(x_ref[...], w_ref[...], preferred_element_type=jnp.float32)
    g1_ref[...] = dis * h0
    dis_ref[...] = dis
    dis2_ref[...] = dis * dis


def _tc_prep(x, W_conv, p0, p1):
    return pl.pallas_call(
        _prep_body,
        grid=(_GRID,),
        in_specs=[
            pl.BlockSpec((_RB, D), lambda i: (i, 0)),
            pl.BlockSpec((D, H), lambda i: (0, 0)),
            pl.BlockSpec((_RB, 1), lambda i: (i, 0)),
            pl.BlockSpec((_RB, 1), lambda i: (i, 0)),
        ],
        out_specs=[
            pl.BlockSpec((_RB, H), lambda i: (i, 0)),
            pl.BlockSpec((_RB, 1), lambda i: (i, 0)),
            pl.BlockSpec((_RB, 1), lambda i: (i, 0)),
        ],
        out_shape=[
            jax.ShapeDtypeStruct((N, H), jnp.float32),
            jax.ShapeDtypeStruct((N, 1), jnp.float32),
            jax.ShapeDtypeStruct((N, 1), jnp.float32),
        ],
    )(x, W_conv, p0, p1)


def _mid_body(p0_ref, p1_ref, g1_ref, dis2_ref, g2_ref):
    g2_ref[...] = dis2_ref[...] * (p0_ref[...] + p1_ref[...] + g1_ref[...])


def _tc_mid(p0, p1, g1, dis2):
    return pl.pallas_call(
        _mid_body,
        grid=(_GRID,),
        in_specs=[
            pl.BlockSpec((_RB, H), lambda i: (i, 0)),
            pl.BlockSpec((_RB, H), lambda i: (i, 0)),
            pl.BlockSpec((_RB, H), lambda i: (i, 0)),
            pl.BlockSpec((_RB, 1), lambda i: (i, 0)),
        ],
        out_specs=pl.BlockSpec((_RB, H), lambda i: (i, 0)),
        out_shape=jax.ShapeDtypeStruct((N, H), jnp.float32),
    )(p0, p1, g1, dis2)


def _out_body(q0_ref, q1_ref, g2_ref, dis_ref, bc_ref, wl_ref, bl_ref, out_ref):
    h2 = dis_ref[...] * (q0_ref[...] + q1_ref[...] + g2_ref[...])
    a = jnp.maximum(h2 + bc_ref[...], 0.0)
    out_ref[...] = (jnp.dot(a, wl_ref[...], preferred_element_type=jnp.float32)
                    + bl_ref[...])


def _tc_out(q0, q1, g2, dis, bc, wl, bl):
    return pl.pallas_call(
        _out_body,
        grid=(_GRID,),
        in_specs=[
            pl.BlockSpec((_RB, H), lambda i: (i, 0)),
            pl.BlockSpec((_RB, H), lambda i: (i, 0)),
            pl.BlockSpec((_RB, H), lambda i: (i, 0)),
            pl.BlockSpec((_RB, 1), lambda i: (i, 0)),
            pl.BlockSpec((1, H), lambda i: (0, 0)),
            pl.BlockSpec((H, OUT), lambda i: (0, 0)),
            pl.BlockSpec((1, OUT), lambda i: (0, 0)),
        ],
        out_specs=pl.BlockSpec((_RB, OUT), lambda i: (i, 0)),
        out_shape=jax.ShapeDtypeStruct((N, OUT), jnp.float32),
    )(q0, q1, g2, dis, bc, wl, bl)


def kernel(x, edge_index, W_conv, b_conv, W_lin, b_lin):
    src = edge_index[0]
    dst = edge_index[1]
    npad_e = EPAD - E
    pad_i = jnp.arange(npad_e, dtype=jnp.int32)
    pad_src = (pad_i * 97) % N
    pad_dst = N + (pad_i % TRASH)
    srcR = jnp.concatenate([src, pad_src]).reshape(NW, CB, CHUNK)
    dstR = jnp.concatenate([dst, pad_dst]).reshape(NW, CB, CHUNK)

    degP = _deg(dstR)
    p0 = degP[:N].reshape(N, 1)
    p1 = degP[NPAD:NPAD + N].reshape(N, 1)
    g1, dis, dis2 = _tc_prep(x, W_conv, p0, p1)

    P = _hop(g1, srcR, dstR)
    g2 = _tc_mid(P[:N], P[NPAD:NPAD + N], g1, dis2)

    Q = _hop(g2, srcR, dstR)
    out = _tc_out(Q[:N], Q[NPAD:NPAD + N], g2, dis,
                  b_conv.reshape(1, H), W_lin, b_lin.reshape(1, OUT))
    return out


# speedup vs baseline: 55.0080x; 1.2305x over previous
import functools

import jax
import jax.numpy as jnp
from jax import lax
from jax.experimental import pallas as pl
from jax.experimental.pallas import tpu as pltpu
from jax.experimental.pallas import tpu_sc as plsc

N = 10000
D = 128
H = 16
OUT = 128
E = 320000

NC = 2
NS = 16
NW = NC * NS
CHUNK = 128
CB = 80
EPAD = NW * CB * CHUNK
NPAD = 10112
TRASH = NPAD - N
RS_ACC = NPAD // NS

_mesh = plsc.VectorSubcoreMesh(core_axis_name="c", subcore_axis_name="s")


def _deg_body(er, out, dst_v, ones_v, tmp_v, deg_s):
    cid = lax.axis_index("c")
    sid = lax.axis_index("s")
    w = cid * NS + sid
    for j in range(RS_ACC // 16):
        tmp_v[pl.ds(j * 16, 16)] = jnp.zeros((16,), jnp.float32)
    pltpu.sync_copy(tmp_v, deg_s.at[pl.ds(sid * RS_ACC, RS_ACC)])
    pltpu.sync_copy(er.at[1, w], dst_v)
    for j in range(CHUNK // 16):
        ones_v[pl.ds(j * 16, 16)] = jnp.ones((16,), jnp.float32)
    plsc.subcore_barrier()

    def step(c, carry):
        pltpu.sync_copy(ones_v, deg_s.at[dst_v.at[c]], add=True)
        return carry

    lax.fori_loop(0, CB, step, 0)
    plsc.subcore_barrier()
    pltpu.sync_copy(deg_s.at[pl.ds(sid * RS_ACC, RS_ACC)], tmp_v)
    pltpu.sync_copy(tmp_v, out.at[pl.ds(cid * NPAD + sid * RS_ACC, RS_ACC)])


_deg = pl.kernel(
    _deg_body,
    out_type=jax.ShapeDtypeStruct((NC * NPAD,), jnp.float32),
    mesh=_mesh,
    scratch_types=[
        pltpu.VMEM((CB, CHUNK), jnp.int32),
        pltpu.VMEM((CHUNK,), jnp.float32),
        pltpu.VMEM((RS_ACC,), jnp.float32),
        pltpu.VMEM_SHARED((NPAD,), jnp.float32),
    ],
)


def _hop_body(g_hbm, er, out, src_v, dst_v, bufa_v, bufb_v, tmp_v, acc_s,
              sema, semb):
    cid = lax.axis_index("c")
    sid = lax.axis_index("s")
    w = cid * NS + sid
    def zstep(j, carry):
        tmp_v[j] = jnp.zeros((16,), jnp.float32)
        return carry

    lax.fori_loop(0, RS_ACC, zstep, 0)
    pltpu.sync_copy(tmp_v, acc_s.at[pl.ds(sid * RS_ACC, RS_ACC), :])
    pltpu.sync_copy(er.at[0, w], src_v)
    pltpu.sync_copy(er.at[1, w], dst_v)
    plsc.subcore_barrier()

    def gath(c, buf, sem):
        return pltpu.async_copy(g_hbm.at[src_v.at[c]], buf, sem)

    def scat(c, buf):
        pltpu.sync_copy(buf, acc_s.at[dst_v.at[c]], add=True)

    gath(0, bufa_v, sema)

    def step(i, carry):
        c = 2 * i
        gath(c + 1, bufb_v, semb)
        pltpu.make_async_copy(g_hbm.at[src_v.at[c]], bufa_v, sema).wait()
        scat(c, bufa_v)
        gath(c + 2, bufa_v, sema)
        pltpu.make_async_copy(g_hbm.at[src_v.at[c]], bufb_v, semb).wait()
        scat(c + 1, bufb_v)
        return carry

    lax.fori_loop(0, CB // 2 - 1, step, 0)
    gath(CB - 1, bufb_v, semb)
    pltpu.make_async_copy(g_hbm.at[src_v.at[0]], bufa_v, sema).wait()
    scat(CB - 2, bufa_v)
    pltpu.make_async_copy(g_hbm.at[src_v.at[0]], bufb_v, semb).wait()
    scat(CB - 1, bufb_v)

    plsc.subcore_barrier()
    pltpu.sync_copy(acc_s.at[pl.ds(sid * RS_ACC, RS_ACC), :], tmp_v)
    pltpu.sync_copy(tmp_v, out.at[pl.ds(cid * NPAD + sid * RS_ACC, RS_ACC), :])


_hop = pl.kernel(
    _hop_body,
    out_type=jax.ShapeDtypeStruct((NC * NPAD, H), jnp.float32),
    mesh=_mesh,
    scratch_types=[
        pltpu.VMEM((CB, CHUNK), jnp.int32),
        pltpu.VMEM((CB, CHUNK), jnp.int32),
        pltpu.VMEM((CHUNK, H), jnp.float32),
        pltpu.VMEM((CHUNK, H), jnp.float32),
        pltpu.VMEM((RS_ACC, H), jnp.float32),
        pltpu.VMEM_SHARED((NPAD, H), jnp.float32),
        pltpu.SemaphoreType.DMA,
        pltpu.SemaphoreType.DMA,
    ],
    compiler_params=pltpu.CompilerParams(use_tc_tiling_on_sc=False),
)


def _prep_body(x_ref, w_ref, degp_ref, g1_ref, dis_ref, dis2_ref):
    deg = degp_ref[0:N] + degp_ref[NPAD:NPAD + N] + 1.0
    dis1 = lax.rsqrt(deg)
    dis = jnp.broadcast_to(dis1.reshape(N, 1), (N, H))
    h0 = jnp.dot(x_ref[...], w_ref[...], preferred_element_type=jnp.float32)
    g1_ref[...] = dis * h0
    dis_ref[...] = dis
    dis2_ref[...] = dis * dis


def _tc_prep(x, W_conv, degP):
    return pl.pallas_call(
        _prep_body,
        out_shape=[
            jax.ShapeDtypeStruct((N, H), jnp.float32),
            jax.ShapeDtypeStruct((N, H), jnp.float32),
            jax.ShapeDtypeStruct((N, H), jnp.float32),
        ],
    )(x, W_conv, degP)


def _mid_body(p_ref, g1_ref, dis2_ref, g2_ref):
    acc = p_ref[0:N, :] + p_ref[NPAD:NPAD + N, :] + g1_ref[...]
    g2_ref[...] = dis2_ref[...] * acc


def _tc_mid(P, g1, dis2):
    return pl.pallas_call(
        _mid_body,
        out_shape=jax.ShapeDtypeStruct((N, H), jnp.float32),
    )(P, g1, dis2)


def _out_body(q_ref, g2_ref, dis_ref, bc_ref, wl_ref, bl_ref, out_ref):
    h2 = dis_ref[...] * (q_ref[0:N, :] + q_ref[NPAD:NPAD + N, :] + g2_ref[...])
    a = jnp.maximum(h2 + bc_ref[...], 0.0)
    out_ref[...] = (jnp.dot(a, wl_ref[...], preferred_element_type=jnp.float32)
                    + bl_ref[...])


def _tc_out(Q, g2, dis, bc, wl, bl):
    return pl.pallas_call(
        _out_body,
        out_shape=jax.ShapeDtypeStruct((N, OUT), jnp.float32),
    )(Q, g2, dis, bc, wl, bl)


def kernel(x, edge_index, W_conv, b_conv, W_lin, b_lin):
    npad_e = EPAD - E
    pad_i = jnp.arange(npad_e, dtype=jnp.int32)
    pad = jnp.stack([(pad_i * 97) % N, N + (pad_i % TRASH)])
    er = jnp.concatenate([edge_index, pad], axis=1).reshape(2, NW, CB, CHUNK)

    degP = _deg(er)
    g1, dis, dis2 = _tc_prep(x, W_conv, degP)

    P = _hop(g1, er)
    g2 = _tc_mid(P, g1, dis2)

    Q = _hop(g2, er)
    out = _tc_out(Q, g2, dis,
                  b_conv.reshape(1, H), W_lin, b_lin.reshape(1, OUT))
    return out


# speedup vs baseline: 65.2284x; 1.1858x over previous
import functools

import jax
import jax.numpy as jnp
from jax import lax
from jax.experimental import pallas as pl
from jax.experimental.pallas import tpu as pltpu
from jax.experimental.pallas import tpu_sc as plsc

N = 10000
D = 128
H = 16
OUT = 128
E = 320000

NC = 2
NS = 16
NW = NC * NS
CHUNK = 256
CB = 40
EPAD = NW * CB * CHUNK
NPAD = 10112
TRASH = NPAD - N
RS_ACC = NPAD // NS

_mesh = plsc.VectorSubcoreMesh(core_axis_name="c", subcore_axis_name="s")


def _deg_body(er, out, dst_v, ones_v, tmp_v, deg_s):
    cid = lax.axis_index("c")
    sid = lax.axis_index("s")
    w = cid * NS + sid
    for j in range(RS_ACC // 16):
        tmp_v[pl.ds(j * 16, 16)] = jnp.zeros((16,), jnp.float32)
    pltpu.sync_copy(tmp_v, deg_s.at[pl.ds(sid * RS_ACC, RS_ACC)])
    pltpu.sync_copy(er.at[1, w], dst_v)
    for j in range(CHUNK // 16):
        ones_v[pl.ds(j * 16, 16)] = jnp.ones((16,), jnp.float32)
    plsc.subcore_barrier()

    def step(c, carry):
        pltpu.sync_copy(ones_v, deg_s.at[dst_v.at[c]], add=True)
        return carry

    lax.fori_loop(0, CB, step, 0)
    plsc.subcore_barrier()
    pltpu.sync_copy(deg_s.at[pl.ds(sid * RS_ACC, RS_ACC)], tmp_v)
    pltpu.sync_copy(tmp_v, out.at[pl.ds(cid * NPAD + sid * RS_ACC, RS_ACC)])


_deg = pl.kernel(
    _deg_body,
    out_type=jax.ShapeDtypeStruct((NC * NPAD,), jnp.float32),
    mesh=_mesh,
    scratch_types=[
        pltpu.VMEM((CB, CHUNK), jnp.int32),
        pltpu.VMEM((CHUNK,), jnp.float32),
        pltpu.VMEM((RS_ACC,), jnp.float32),
        pltpu.VMEM_SHARED((NPAD,), jnp.float32),
    ],
    compiler_params=pltpu.CompilerParams(use_tc_tiling_on_sc=False),
)


def _hop_body(g_hbm, er, out, src_v, dst_v, bufa_v, bufb_v, tmp_v, acc_s,
              sema, semb):
    cid = lax.axis_index("c")
    sid = lax.axis_index("s")
    w = cid * NS + sid
    def zstep(j, carry):
        tmp_v[j] = jnp.zeros((16,), jnp.float32)
        return carry

    lax.fori_loop(0, RS_ACC, zstep, 0)
    pltpu.sync_copy(tmp_v, acc_s.at[pl.ds(sid * RS_ACC, RS_ACC), :])
    pltpu.sync_copy(er.at[0, w], src_v)
    pltpu.sync_copy(er.at[1, w], dst_v)
    plsc.subcore_barrier()

    def gath(c, buf, sem):
        return pltpu.async_copy(g_hbm.at[src_v.at[c]], buf, sem)

    def scat(c, buf):
        pltpu.sync_copy(buf, acc_s.at[dst_v.at[c]], add=True)

    gath(0, bufa_v, sema)

    def step(i, carry):
        c = 2 * i
        gath(c + 1, bufb_v, semb)
        pltpu.make_async_copy(g_hbm.at[src_v.at[c]], bufa_v, sema).wait()
        scat(c, bufa_v)
        gath(c + 2, bufa_v, sema)
        pltpu.make_async_copy(g_hbm.at[src_v.at[c]], bufb_v, semb).wait()
        scat(c + 1, bufb_v)
        return carry

    lax.fori_loop(0, CB // 2 - 1, step, 0)
    gath(CB - 1, bufb_v, semb)
    pltpu.make_async_copy(g_hbm.at[src_v.at[0]], bufa_v, sema).wait()
    scat(CB - 2, bufa_v)
    pltpu.make_async_copy(g_hbm.at[src_v.at[0]], bufb_v, semb).wait()
    scat(CB - 1, bufb_v)

    plsc.subcore_barrier()
    pltpu.sync_copy(acc_s.at[pl.ds(sid * RS_ACC, RS_ACC), :], tmp_v)
    pltpu.sync_copy(tmp_v, out.at[pl.ds(cid * NPAD + sid * RS_ACC, RS_ACC), :])


_hop = pl.kernel(
    _hop_body,
    out_type=jax.ShapeDtypeStruct((NC * NPAD, H), jnp.float32),
    mesh=_mesh,
    scratch_types=[
        pltpu.VMEM((CB, CHUNK), jnp.int32),
        pltpu.VMEM((CB, CHUNK), jnp.int32),
        pltpu.VMEM((CHUNK, H), jnp.float32),
        pltpu.VMEM((CHUNK, H), jnp.float32),
        pltpu.VMEM((RS_ACC, H), jnp.float32),
        pltpu.VMEM_SHARED((NPAD, H), jnp.float32),
        pltpu.SemaphoreType.DMA,
        pltpu.SemaphoreType.DMA,
    ],
    compiler_params=pltpu.CompilerParams(use_tc_tiling_on_sc=False),
)


def _prep_body(x_ref, w_ref, degp_ref, g1_ref, dis_ref, dis2_ref):
    deg = degp_ref[0:N] + degp_ref[NPAD:NPAD + N] + 1.0
    dis1 = lax.rsqrt(deg)
    dis = jnp.broadcast_to(dis1.reshape(N, 1), (N, H))
    h0 = jnp.dot(x_ref[...], w_ref[...], preferred_element_type=jnp.float32)
    g1_ref[...] = dis * h0
    dis_ref[...] = dis
    dis2_ref[...] = dis * dis


def _tc_prep(x, W_conv, degP):
    return pl.pallas_call(
        _prep_body,
        out_shape=[
            jax.ShapeDtypeStruct((N, H), jnp.float32),
            jax.ShapeDtypeStruct((N, H), jnp.float32),
            jax.ShapeDtypeStruct((N, H), jnp.float32),
        ],
    )(x, W_conv, degP)


def _mid_body(p_ref, g1_ref, dis2_ref, g2_ref):
    acc = p_ref[0:N, :] + p_ref[NPAD:NPAD + N, :] + g1_ref[...]
    g2_ref[...] = dis2_ref[...] * acc


def _tc_mid(P, g1, dis2):
    return pl.pallas_call(
        _mid_body,
        out_shape=jax.ShapeDtypeStruct((N, H), jnp.float32),
    )(P, g1, dis2)


def _out_body(q_ref, g2_ref, dis_ref, bc_ref, wl_ref, bl_ref, out_ref):
    h2 = dis_ref[...] * (q_ref[0:N, :] + q_ref[NPAD:NPAD + N, :] + g2_ref[...])
    a = jnp.maximum(h2 + bc_ref[...], 0.0)
    out_ref[...] = (jnp.dot(a, wl_ref[...], preferred_element_type=jnp.float32)
                    + bl_ref[...])


def _tc_out(Q, g2, dis, bc, wl, bl):
    return pl.pallas_call(
        _out_body,
        out_shape=jax.ShapeDtypeStruct((N, OUT), jnp.float32),
    )(Q, g2, dis, bc, wl, bl)


def kernel(x, edge_index, W_conv, b_conv, W_lin, b_lin):
    npad_e = EPAD - E
    pad_i = jnp.arange(npad_e, dtype=jnp.int32)
    pad = jnp.stack([(pad_i * 97) % N, N + (pad_i % TRASH)])
    er = jnp.concatenate([edge_index, pad], axis=1).reshape(2, NW, CB, CHUNK)

    degP = _deg(er)
    g1, dis, dis2 = _tc_prep(x, W_conv, degP)

    P = _hop(g1, er)
    g2 = _tc_mid(P, g1, dis2)

    Q = _hop(g2, er)
    out = _tc_out(Q, g2, dis,
                  b_conv.reshape(1, H), W_lin, b_lin.reshape(1, OUT))
    return out


# speedup vs baseline: 71.4320x; 1.0951x over previous
import functools

import jax
import jax.numpy as jnp
from jax import lax
from jax.experimental import pallas as pl
from jax.experimental.pallas import tpu as pltpu
from jax.experimental.pallas import tpu_sc as plsc

N = 10000
D = 128
H = 16
OUT = 128
E = 320000

NC = 2
NS = 16
NW = NC * NS
CHUNK = 512
CB = 20
EPAD = NW * CB * CHUNK
NPAD = 10112
TRASH = NPAD - N
RS_ACC = NPAD // NS

_mesh = plsc.VectorSubcoreMesh(core_axis_name="c", subcore_axis_name="s")


def _deg_body(er, out, dst_v, ones_v, tmp_v, deg_s):
    cid = lax.axis_index("c")
    sid = lax.axis_index("s")
    w = cid * NS + sid
    for j in range(RS_ACC // 16):
        tmp_v[pl.ds(j * 16, 16)] = jnp.zeros((16,), jnp.float32)
    pltpu.sync_copy(tmp_v, deg_s.at[pl.ds(sid * RS_ACC, RS_ACC)])
    pltpu.sync_copy(er.at[1, w], dst_v)
    for j in range(CHUNK // 16):
        ones_v[pl.ds(j * 16, 16)] = jnp.ones((16,), jnp.float32)
    plsc.subcore_barrier()

    def step(c, carry):
        pltpu.sync_copy(ones_v, deg_s.at[dst_v.at[c]], add=True)
        return carry

    lax.fori_loop(0, CB, step, 0)
    plsc.subcore_barrier()
    pltpu.sync_copy(deg_s.at[pl.ds(sid * RS_ACC, RS_ACC)], tmp_v)
    pltpu.sync_copy(tmp_v, out.at[pl.ds(cid * NPAD + sid * RS_ACC, RS_ACC)])


_deg = pl.kernel(
    _deg_body,
    out_type=jax.ShapeDtypeStruct((NC * NPAD,), jnp.float32),
    mesh=_mesh,
    scratch_types=[
        pltpu.VMEM((CB, CHUNK), jnp.int32),
        pltpu.VMEM((CHUNK,), jnp.float32),
        pltpu.VMEM((RS_ACC,), jnp.float32),
        pltpu.VMEM_SHARED((NPAD,), jnp.float32),
    ],
    compiler_params=pltpu.CompilerParams(use_tc_tiling_on_sc=False),
)


def _hop_body(g_hbm, er, out, src_v, dst_v, bufa_v, bufb_v, tmp_v, acc_s,
              sema, semb):
    cid = lax.axis_index("c")
    sid = lax.axis_index("s")
    w = cid * NS + sid
    def zstep(j, carry):
        tmp_v[j] = jnp.zeros((16,), jnp.float32)
        return carry

    lax.fori_loop(0, RS_ACC, zstep, 0)
    pltpu.sync_copy(tmp_v, acc_s.at[pl.ds(sid * RS_ACC, RS_ACC), :])
    pltpu.sync_copy(er.at[0, w], src_v)
    pltpu.sync_copy(er.at[1, w], dst_v)
    plsc.subcore_barrier()

    def gath(c, buf, sem):
        return pltpu.async_copy(g_hbm.at[src_v.at[c]], buf, sem)

    def scat(c, buf):
        pltpu.sync_copy(buf, acc_s.at[dst_v.at[c]], add=True)

    gath(0, bufa_v, sema)

    def step(i, carry):
        c = 2 * i
        gath(c + 1, bufb_v, semb)
        pltpu.make_async_copy(g_hbm.at[src_v.at[c]], bufa_v, sema).wait()
        scat(c, bufa_v)
        gath(c + 2, bufa_v, sema)
        pltpu.make_async_copy(g_hbm.at[src_v.at[c]], bufb_v, semb).wait()
        scat(c + 1, bufb_v)
        return carry

    lax.fori_loop(0, CB // 2 - 1, step, 0)
    gath(CB - 1, bufb_v, semb)
    pltpu.make_async_copy(g_hbm.at[src_v.at[0]], bufa_v, sema).wait()
    scat(CB - 2, bufa_v)
    pltpu.make_async_copy(g_hbm.at[src_v.at[0]], bufb_v, semb).wait()
    scat(CB - 1, bufb_v)

    plsc.subcore_barrier()
    pltpu.sync_copy(acc_s.at[pl.ds(sid * RS_ACC, RS_ACC), :], tmp_v)
    pltpu.sync_copy(tmp_v, out.at[pl.ds(cid * NPAD + sid * RS_ACC, RS_ACC), :])


_hop = pl.kernel(
    _hop_body,
    out_type=jax.ShapeDtypeStruct((NC * NPAD, H), jnp.float32),
    mesh=_mesh,
    scratch_types=[
        pltpu.VMEM((CB, CHUNK), jnp.int32),
        pltpu.VMEM((CB, CHUNK), jnp.int32),
        pltpu.VMEM((CHUNK, H), jnp.float32),
        pltpu.VMEM((CHUNK, H), jnp.float32),
        pltpu.VMEM((RS_ACC, H), jnp.float32),
        pltpu.VMEM_SHARED((NPAD, H), jnp.float32),
        pltpu.SemaphoreType.DMA,
        pltpu.SemaphoreType.DMA,
    ],
    compiler_params=pltpu.CompilerParams(use_tc_tiling_on_sc=False),
)


def _prep_body(x_ref, w_ref, degp_ref, g1_ref, dis_ref, dis2_ref):
    deg = degp_ref[0:N] + degp_ref[NPAD:NPAD + N] + 1.0
    dis1 = lax.rsqrt(deg)
    dis = jnp.broadcast_to(dis1.reshape(N, 1), (N, H))
    h0 = jnp.dot(x_ref[...], w_ref[...], preferred_element_type=jnp.float32)
    g1_ref[...] = dis * h0
    dis_ref[...] = dis
    dis2_ref[...] = dis * dis


def _tc_prep(x, W_conv, degP):
    return pl.pallas_call(
        _prep_body,
        out_shape=[
            jax.ShapeDtypeStruct((N, H), jnp.float32),
            jax.ShapeDtypeStruct((N, H), jnp.float32),
            jax.ShapeDtypeStruct((N, H), jnp.float32),
        ],
    )(x, W_conv, degP)


def _mid_body(p_ref, g1_ref, dis2_ref, g2_ref):
    acc = p_ref[0:N, :] + p_ref[NPAD:NPAD + N, :] + g1_ref[...]
    g2_ref[...] = dis2_ref[...] * acc


def _tc_mid(P, g1, dis2):
    return pl.pallas_call(
        _mid_body,
        out_shape=jax.ShapeDtypeStruct((N, H), jnp.float32),
    )(P, g1, dis2)


def _out_body(q_ref, g2_ref, dis_ref, bc_ref, wl_ref, bl_ref, out_ref):
    h2 = dis_ref[...] * (q_ref[0:N, :] + q_ref[NPAD:NPAD + N, :] + g2_ref[...])
    a = jnp.maximum(h2 + bc_ref[...], 0.0)
    out_ref[...] = (jnp.dot(a, wl_ref[...], preferred_element_type=jnp.float32)
                    + bl_ref[...])


def _tc_out(Q, g2, dis, bc, wl, bl):
    return pl.pallas_call(
        _out_body,
        out_shape=jax.ShapeDtypeStruct((N, OUT), jnp.float32),
    )(Q, g2, dis, bc, wl, bl)


def kernel(x, edge_index, W_conv, b_conv, W_lin, b_lin):
    npad_e = EPAD - E
    pad_i = jnp.arange(npad_e, dtype=jnp.int32)
    pad = jnp.stack([(pad_i * 97) % N, N + (pad_i % TRASH)])
    er = jnp.concatenate([edge_index, pad], axis=1).reshape(2, NW, CB, CHUNK)

    degP = _deg(er)
    g1, dis, dis2 = _tc_prep(x, W_conv, degP)

    P = _hop(g1, er)
    g2 = _tc_mid(P, g1, dis2)

    Q = _hop(g2, er)
    out = _tc_out(Q, g2, dis,
                  b_conv.reshape(1, H), W_lin, b_lin.reshape(1, OUT))
    return out


# speedup vs baseline: 72.9038x; 1.0206x over previous
import functools

import jax
import jax.numpy as jnp
from jax import lax
from jax.experimental import pallas as pl
from jax.experimental.pallas import tpu as pltpu
from jax.experimental.pallas import tpu_sc as plsc

N = 10000
D = 128
H = 16
OUT = 128
E = 320000

NC = 2
NS = 16
NW = NC * NS
CHUNK = 512
CB = 20
EPAD = NW * CB * CHUNK
NPAD = 10112
TRASH = NPAD - N
RS_ACC = NPAD // NS

_mesh = plsc.VectorSubcoreMesh(core_axis_name="c", subcore_axis_name="s")


def _deg_body(er, out, dst_v, ones_v, tmp_v, deg_s):
    cid = lax.axis_index("c")
    sid = lax.axis_index("s")
    w = cid * NS + sid
    for j in range(RS_ACC // 16):
        tmp_v[pl.ds(j * 16, 16)] = jnp.zeros((16,), jnp.float32)
    pltpu.sync_copy(tmp_v, deg_s.at[pl.ds(sid * RS_ACC, RS_ACC)])
    pltpu.sync_copy(er.at[1, w], dst_v)
    for j in range(CHUNK // 16):
        ones_v[pl.ds(j * 16, 16)] = jnp.ones((16,), jnp.float32)
    plsc.subcore_barrier()

    def step(c, carry):
        pltpu.sync_copy(ones_v, deg_s.at[dst_v.at[c]], add=True)
        return carry

    lax.fori_loop(0, CB, step, 0)
    plsc.subcore_barrier()
    pltpu.sync_copy(deg_s.at[pl.ds(sid * RS_ACC, RS_ACC)], tmp_v)
    pltpu.sync_copy(tmp_v, out.at[pl.ds(cid * NPAD + sid * RS_ACC, RS_ACC)])


_deg = pl.kernel(
    _deg_body,
    out_type=jax.ShapeDtypeStruct((NC * NPAD,), jnp.float32),
    mesh=_mesh,
    scratch_types=[
        pltpu.VMEM((CB, CHUNK), jnp.int32),
        pltpu.VMEM((CHUNK,), jnp.float32),
        pltpu.VMEM((RS_ACC,), jnp.float32),
        pltpu.VMEM_SHARED((NPAD,), jnp.float32),
    ],
    compiler_params=pltpu.CompilerParams(use_tc_tiling_on_sc=False),
)


def _hop_body(g_hbm, er, out, src_v, dst_v, bufa_v, bufb_v, tmp_v, acc_s,
              tab_s, sema, semb):
    cid = lax.axis_index("c")
    sid = lax.axis_index("s")
    w = cid * NS + sid
    def zstep(j, carry):
        tmp_v[j] = jnp.zeros((16,), jnp.float32)
        return carry

    lax.fori_loop(0, RS_ACC, zstep, 0)
    pltpu.sync_copy(tmp_v, acc_s.at[pl.ds(sid * RS_ACC, RS_ACC), :])
    pltpu.sync_copy(g_hbm.at[pl.ds(sid * RS_ACC, RS_ACC), :], tmp_v)
    pltpu.sync_copy(tmp_v, tab_s.at[pl.ds(sid * RS_ACC, RS_ACC), :])
    pltpu.sync_copy(er.at[0, w], src_v)
    pltpu.sync_copy(er.at[1, w], dst_v)
    plsc.subcore_barrier()

    def gath(c, buf, sem):
        return pltpu.async_copy(tab_s.at[src_v.at[c]], buf, sem)

    def scat(c, buf):
        pltpu.sync_copy(buf, acc_s.at[dst_v.at[c]], add=True)

    gath(0, bufa_v, sema)

    def step(i, carry):
        c = 2 * i
        gath(c + 1, bufb_v, semb)
        pltpu.make_async_copy(tab_s.at[src_v.at[c]], bufa_v, sema).wait()
        scat(c, bufa_v)
        gath(c + 2, bufa_v, sema)
        pltpu.make_async_copy(tab_s.at[src_v.at[c]], bufb_v, semb).wait()
        scat(c + 1, bufb_v)
        return carry

    lax.fori_loop(0, CB // 2 - 1, step, 0)
    gath(CB - 1, bufb_v, semb)
    pltpu.make_async_copy(tab_s.at[src_v.at[0]], bufa_v, sema).wait()
    scat(CB - 2, bufa_v)
    pltpu.make_async_copy(tab_s.at[src_v.at[0]], bufb_v, semb).wait()
    scat(CB - 1, bufb_v)

    plsc.subcore_barrier()
    pltpu.sync_copy(acc_s.at[pl.ds(sid * RS_ACC, RS_ACC), :], tmp_v)
    pltpu.sync_copy(tmp_v, out.at[pl.ds(cid * NPAD + sid * RS_ACC, RS_ACC), :])


_hop = pl.kernel(
    _hop_body,
    out_type=jax.ShapeDtypeStruct((NC * NPAD, H), jnp.float32),
    mesh=_mesh,
    scratch_types=[
        pltpu.VMEM((CB, CHUNK), jnp.int32),
        pltpu.VMEM((CB, CHUNK), jnp.int32),
        pltpu.VMEM((CHUNK, H), jnp.float32),
        pltpu.VMEM((CHUNK, H), jnp.float32),
        pltpu.VMEM((RS_ACC, H), jnp.float32),
        pltpu.VMEM_SHARED((NPAD, H), jnp.float32),
        pltpu.VMEM_SHARED((NPAD, H), jnp.float32),
        pltpu.SemaphoreType.DMA,
        pltpu.SemaphoreType.DMA,
    ],
    compiler_params=pltpu.CompilerParams(use_tc_tiling_on_sc=False),
)


def _prep_body(x_ref, w_ref, degp_ref, g1_ref, dis_ref, dis2_ref):
    deg = degp_ref[0:N] + degp_ref[NPAD:NPAD + N] + 1.0
    dis1 = lax.rsqrt(deg)
    dis = jnp.broadcast_to(dis1.reshape(N, 1), (N, H))
    h0 = jnp.dot(x_ref[...], w_ref[...], preferred_element_type=jnp.float32)
    g1_ref[0:N, :] = dis * h0
    g1_ref[N:NPAD, :] = jnp.zeros((TRASH, H), jnp.float32)
    dis_ref[...] = dis
    dis2_ref[...] = dis * dis


def _tc_prep(x, W_conv, degP):
    return pl.pallas_call(
        _prep_body,
        out_shape=[
            jax.ShapeDtypeStruct((NPAD, H), jnp.float32),
            jax.ShapeDtypeStruct((N, H), jnp.float32),
            jax.ShapeDtypeStruct((N, H), jnp.float32),
        ],
    )(x, W_conv, degP)


def _mid_body(p_ref, g1_ref, dis2_ref, g2_ref):
    acc = p_ref[0:N, :] + p_ref[NPAD:NPAD + N, :] + g1_ref[0:N, :]
    g2_ref[0:N, :] = dis2_ref[...] * acc
    g2_ref[N:NPAD, :] = jnp.zeros((TRASH, H), jnp.float32)


def _tc_mid(P, g1, dis2):
    return pl.pallas_call(
        _mid_body,
        out_shape=jax.ShapeDtypeStruct((NPAD, H), jnp.float32),
    )(P, g1, dis2)


def _out_body(q_ref, g2_ref, dis_ref, bc_ref, wl_ref, bl_ref, out_ref):
    h2 = dis_ref[...] * (q_ref[0:N, :] + q_ref[NPAD:NPAD + N, :]
                         + g2_ref[0:N, :])
    a = jnp.maximum(h2 + bc_ref[...], 0.0)
    out_ref[...] = (jnp.dot(a, wl_ref[...], preferred_element_type=jnp.float32)
                    + bl_ref[...])


def _tc_out(Q, g2, dis, bc, wl, bl):
    return pl.pallas_call(
        _out_body,
        out_shape=jax.ShapeDtypeStruct((N, OUT), jnp.float32),
    )(Q, g2, dis, bc, wl, bl)


def kernel(x, edge_index, W_conv, b_conv, W_lin, b_lin):
    npad_e = EPAD - E
    pad_i = jnp.arange(npad_e, dtype=jnp.int32)
    pad = jnp.stack([(pad_i * 97) % N, N + (pad_i % TRASH)])
    er = jnp.concatenate([edge_index, pad], axis=1).reshape(2, NW, CB, CHUNK)

    degP = _deg(er)
    g1, dis, dis2 = _tc_prep(x, W_conv, degP)

    P = _hop(g1, er)
    g2 = _tc_mid(P, g1, dis2)

    Q = _hop(g2, er)
    out = _tc_out(Q, g2, dis,
                  b_conv.reshape(1, H), W_lin, b_lin.reshape(1, OUT))
    return out


# speedup vs baseline: 75.3457x; 1.0335x over previous
import functools

import jax
import jax.numpy as jnp
from jax import lax
from jax.experimental import pallas as pl
from jax.experimental.pallas import tpu as pltpu
from jax.experimental.pallas import tpu_sc as plsc

N = 10000
D = 128
H = 16
OUT = 128
E = 320000

NC = 2
NS = 16
NW = NC * NS
CHUNK = 1024
CB = 10
EPAD = NW * CB * CHUNK
NPAD = 10112
TRASH = NPAD - N
RS_ACC = NPAD // NS

_mesh = plsc.VectorSubcoreMesh(core_axis_name="c", subcore_axis_name="s")


def _deg_body(er, out, dst_v, ones_v, tmp_v, deg_s):
    cid = lax.axis_index("c")
    sid = lax.axis_index("s")
    w = cid * NS + sid
    for j in range(RS_ACC // 16):
        tmp_v[pl.ds(j * 16, 16)] = jnp.zeros((16,), jnp.float32)
    pltpu.sync_copy(tmp_v, deg_s.at[pl.ds(sid * RS_ACC, RS_ACC)])
    pltpu.sync_copy(er.at[1, w], dst_v)
    for j in range(CHUNK // 16):
        ones_v[pl.ds(j * 16, 16)] = jnp.ones((16,), jnp.float32)
    plsc.subcore_barrier()

    def step(c, carry):
        pltpu.sync_copy(ones_v, deg_s.at[dst_v.at[c]], add=True)
        return carry

    lax.fori_loop(0, CB, step, 0)
    plsc.subcore_barrier()
    pltpu.sync_copy(deg_s.at[pl.ds(sid * RS_ACC, RS_ACC)], tmp_v)
    pltpu.sync_copy(tmp_v, out.at[pl.ds(cid * NPAD + sid * RS_ACC, RS_ACC)])


_deg = pl.kernel(
    _deg_body,
    out_type=jax.ShapeDtypeStruct((NC * NPAD,), jnp.float32),
    mesh=_mesh,
    scratch_types=[
        pltpu.VMEM((CB, CHUNK), jnp.int32),
        pltpu.VMEM((CHUNK,), jnp.float32),
        pltpu.VMEM((RS_ACC,), jnp.float32),
        pltpu.VMEM_SHARED((NPAD,), jnp.float32),
    ],
    compiler_params=pltpu.CompilerParams(use_tc_tiling_on_sc=False),
)


def _hop_body(g_hbm, er, out, src_v, dst_v, bufa_v, bufb_v, tmp_v, acc_s,
              tab_s, sema, semb):
    cid = lax.axis_index("c")
    sid = lax.axis_index("s")
    w = cid * NS + sid
    def zstep(j, carry):
        tmp_v[j] = jnp.zeros((16,), jnp.float32)
        return carry

    lax.fori_loop(0, RS_ACC, zstep, 0)
    pltpu.sync_copy(tmp_v, acc_s.at[pl.ds(sid * RS_ACC, RS_ACC), :])
    pltpu.sync_copy(g_hbm.at[pl.ds(sid * RS_ACC, RS_ACC), :], tmp_v)
    pltpu.sync_copy(tmp_v, tab_s.at[pl.ds(sid * RS_ACC, RS_ACC), :])
    pltpu.sync_copy(er.at[0, w], src_v)
    pltpu.sync_copy(er.at[1, w], dst_v)
    plsc.subcore_barrier()

    def gath(c, buf, sem):
        return pltpu.async_copy(tab_s.at[src_v.at[c]], buf, sem)

    def scat(c, buf):
        pltpu.sync_copy(buf, acc_s.at[dst_v.at[c]], add=True)

    gath(0, bufa_v, sema)

    def step(i, carry):
        c = 2 * i
        gath(c + 1, bufb_v, semb)
        pltpu.make_async_copy(tab_s.at[src_v.at[c]], bufa_v, sema).wait()
        scat(c, bufa_v)
        gath(c + 2, bufa_v, sema)
        pltpu.make_async_copy(tab_s.at[src_v.at[c]], bufb_v, semb).wait()
        scat(c + 1, bufb_v)
        return carry

    lax.fori_loop(0, CB // 2 - 1, step, 0)
    gath(CB - 1, bufb_v, semb)
    pltpu.make_async_copy(tab_s.at[src_v.at[0]], bufa_v, sema).wait()
    scat(CB - 2, bufa_v)
    pltpu.make_async_copy(tab_s.at[src_v.at[0]], bufb_v, semb).wait()
    scat(CB - 1, bufb_v)

    plsc.subcore_barrier()
    pltpu.sync_copy(acc_s.at[pl.ds(sid * RS_ACC, RS_ACC), :], tmp_v)
    pltpu.sync_copy(tmp_v, out.at[pl.ds(cid * NPAD + sid * RS_ACC, RS_ACC), :])


_hop = pl.kernel(
    _hop_body,
    out_type=jax.ShapeDtypeStruct((NC * NPAD, H), jnp.float32),
    mesh=_mesh,
    scratch_types=[
        pltpu.VMEM((CB, CHUNK), jnp.int32),
        pltpu.VMEM((CB, CHUNK), jnp.int32),
        pltpu.VMEM((CHUNK, H), jnp.float32),
        pltpu.VMEM((CHUNK, H), jnp.float32),
        pltpu.VMEM((RS_ACC, H), jnp.float32),
        pltpu.VMEM_SHARED((NPAD, H), jnp.float32),
        pltpu.VMEM_SHARED((NPAD, H), jnp.float32),
        pltpu.SemaphoreType.DMA,
        pltpu.SemaphoreType.DMA,
    ],
    compiler_params=pltpu.CompilerParams(use_tc_tiling_on_sc=False),
)


def _prep_body(x_ref, w_ref, degp_ref, g1_ref, dis_ref, dis2_ref):
    deg = degp_ref[0:N] + degp_ref[NPAD:NPAD + N] + 1.0
    dis1 = lax.rsqrt(deg)
    dis = jnp.broadcast_to(dis1.reshape(N, 1), (N, H))
    h0 = jnp.dot(x_ref[...], w_ref[...], preferred_element_type=jnp.float32)
    g1_ref[0:N, :] = dis * h0
    g1_ref[N:NPAD, :] = jnp.zeros((TRASH, H), jnp.float32)
    dis_ref[...] = dis
    dis2_ref[...] = dis * dis


def _tc_prep(x, W_conv, degP):
    return pl.pallas_call(
        _prep_body,
        out_shape=[
            jax.ShapeDtypeStruct((NPAD, H), jnp.float32),
            jax.ShapeDtypeStruct((N, H), jnp.float32),
            jax.ShapeDtypeStruct((N, H), jnp.float32),
        ],
    )(x, W_conv, degP)


def _mid_body(p_ref, g1_ref, dis2_ref, g2_ref):
    acc = p_ref[0:N, :] + p_ref[NPAD:NPAD + N, :] + g1_ref[0:N, :]
    g2_ref[0:N, :] = dis2_ref[...] * acc
    g2_ref[N:NPAD, :] = jnp.zeros((TRASH, H), jnp.float32)


def _tc_mid(P, g1, dis2):
    return pl.pallas_call(
        _mid_body,
        out_shape=jax.ShapeDtypeStruct((NPAD, H), jnp.float32),
    )(P, g1, dis2)


def _out_body(q_ref, g2_ref, dis_ref, bc_ref, wl_ref, bl_ref, out_ref):
    h2 = dis_ref[...] * (q_ref[0:N, :] + q_ref[NPAD:NPAD + N, :]
                         + g2_ref[0:N, :])
    a = jnp.maximum(h2 + bc_ref[...], 0.0)
    out_ref[...] = (jnp.dot(a, wl_ref[...], preferred_element_type=jnp.float32)
                    + bl_ref[...])


def _tc_out(Q, g2, dis, bc, wl, bl):
    return pl.pallas_call(
        _out_body,
        out_shape=jax.ShapeDtypeStruct((N, OUT), jnp.float32),
    )(Q, g2, dis, bc, wl, bl)


def kernel(x, edge_index, W_conv, b_conv, W_lin, b_lin):
    npad_e = EPAD - E
    pad_i = jnp.arange(npad_e, dtype=jnp.int32)
    pad = jnp.stack([(pad_i * 97) % N, N + (pad_i % TRASH)])
    er = jnp.concatenate([edge_index, pad], axis=1).reshape(2, NW, CB, CHUNK)

    degP = _deg(er)
    g1, dis, dis2 = _tc_prep(x, W_conv, degP)

    P = _hop(g1, er)
    g2 = _tc_mid(P, g1, dis2)

    Q = _hop(g2, er)
    out = _tc_out(Q, g2, dis,
                  b_conv.reshape(1, H), W_lin, b_lin.reshape(1, OUT))
    return out
